# Initial kernel scaffold; baseline (speedup 1.0000x reference)
import jax
import jax.numpy as jnp
from jax.experimental import pallas as pl


def kernel(hidden_states, weight):
    raise NotImplementedError("write your pallas kernel here")


# speedup vs baseline: 1.4018x; 1.4018x over previous
import functools

import jax
import jax.numpy as jnp
from jax.experimental import pallas as pl
from jax.experimental.pallas import tpu as pltpu

TOP_K = 8
N_EXPERTS = 64
ALPHA = 0.001
HIDDEN = 2048

BLK = 512


def _gate_kernel(x_ref, wt_ref, idx_ref, w_ref, aux_ref, prob_acc, load_acc,
                 *, nblocks, n_rows):
    i = pl.program_id(0)

    @pl.when(i == 0)
    def _init():
        prob_acc[...] = jnp.zeros_like(prob_acc)
        load_acc[...] = jnp.zeros_like(load_acc)

    logits = jnp.dot(x_ref[...], wt_ref[...],
                     preferred_element_type=jnp.float32,
                     precision=jax.lax.Precision.DEFAULT)
    scores = jax.nn.sigmoid(logits)

    row_sum = jnp.sum(scores, axis=1, keepdims=True)
    prob_acc[...] += jnp.sum(scores / (row_sum + 1e-9), axis=0, keepdims=True)

    iota = jax.lax.broadcasted_iota(jnp.int32, scores.shape, 1)
    work = scores
    sel_any = jnp.zeros(scores.shape, jnp.float32)
    vals, idxs = [], []
    for _ in range(TOP_K):
        m = jnp.max(work, axis=1, keepdims=True)
        amx = jnp.argmax(work, axis=1).astype(jnp.int32)[:, None]
        mask = iota == amx
        work = jnp.where(mask, -1.0, work)
        sel_any = sel_any + mask.astype(jnp.float32)
        vals.append(m)
        idxs.append(amx)
    load_acc[...] += jnp.sum(sel_any, axis=0, keepdims=True)

    topv = jnp.concatenate(vals, axis=1)
    denom = jnp.sum(topv, axis=1, keepdims=True) + 1e-9
    w_ref[...] = topv / denom
    idx_ref[...] = jnp.concatenate(idxs, axis=1)

    @pl.when(i == nblocks - 1)
    def _fin():
        load = load_acc[...] / (n_rows * TOP_K)
        prob = prob_acc[...] / n_rows
        prob = prob / (jnp.sum(prob) + 1e-9)
        aux = ALPHA * jnp.sum(load * prob) * N_EXPERTS
        aux_ref[...] = jnp.full((1, 1), aux, jnp.float32)


def kernel(hidden_states, weight):
    B, S, H = hidden_states.shape
    n = B * S
    x = hidden_states.reshape(n, H)
    wt = weight.T
    nblocks = n // BLK

    idx, w, aux = pl.pallas_call(
        functools.partial(_gate_kernel, nblocks=nblocks, n_rows=n),
        grid=(nblocks,),
        in_specs=[
            pl.BlockSpec((BLK, H), lambda i: (i, 0)),
            pl.BlockSpec((H, N_EXPERTS), lambda i: (0, 0)),
        ],
        out_specs=[
            pl.BlockSpec((BLK, TOP_K), lambda i: (i, 0)),
            pl.BlockSpec((BLK, TOP_K), lambda i: (i, 0)),
            pl.BlockSpec((1, 1), lambda i: (0, 0)),
        ],
        out_shape=[
            jax.ShapeDtypeStruct((n, TOP_K), jnp.int32),
            jax.ShapeDtypeStruct((n, TOP_K), jnp.float32),
            jax.ShapeDtypeStruct((1, 1), jnp.float32),
        ],
        scratch_shapes=[
            pltpu.VMEM((1, N_EXPERTS), jnp.float32),
            pltpu.VMEM((1, N_EXPERTS), jnp.float32),
        ],
    )(x, wt)
    return idx, w, aux[0, 0]
